# baseline (speedup 1.0000x reference)
import functools
import jax
import jax.numpy as jnp
from jax import lax
from jax.experimental import pallas as pl
from jax.experimental.pallas import tpu as pltpu
from jax.experimental.pallas import tpu_sc as plsc

_NBLK = 512
_EPS_BN = 1e-3
_BIG = 3.0e38
_NW = 32
_CH = 64


def _nn_body(ut_ref, kn_ref, kf_ref, w1_ref, idx_ref, wts_ref, g_ref):
    b = pl.program_id(0)
    j = pl.program_id(1)
    ut = ut_ref[0]
    kn = kn_ref[0]
    M = kn.shape[0]
    N = ut.shape[1]

    un2 = jnp.sum(ut * ut, axis=0, keepdims=True)
    kn2 = jnp.sum(kn * kn, axis=1, keepdims=True)
    cross = jax.lax.dot_general(
        kn.astype(jnp.bfloat16), ut.astype(jnp.bfloat16),
        (((1,), (0,)), ((), ())),
        preferred_element_type=jnp.float32)
    d2 = jnp.maximum(kn2 + un2 - 2.0 * cross, 0.0)

    iota_f = lax.broadcasted_iota(jnp.int32, (M, N), 0).astype(jnp.float32)
    d2 = jnp.where(d2 == 0.0, iota_f * 1e-30, d2)

    iota_m = lax.broadcasted_iota(jnp.int32, (2, M), 1)
    hilo = jnp.where(lax.broadcasted_iota(jnp.int32, (2, M), 0) == 0,
                     iota_m // 64, iota_m % 64).astype(jnp.bfloat16)

    recips = []
    iks = []
    d2w = d2
    for _ in range(3):
        mk = jnp.min(d2w, axis=0, keepdims=True)
        recips.append(1.0 / (mk + 1e-8))
        sel = d2w == mk
        hl = jax.lax.dot_general(
            hilo, sel.astype(jnp.bfloat16), (((1,), (0,)), ((), ())),
            preferred_element_type=jnp.float32)
        iks.append(hl[0:1, :] * 64.0 + hl[1:2, :])
        d2w = jnp.where(sel, _BIG, d2w)

    norm = recips[0] + recips[1] + recips[2]
    for k in range(3):
        idx_ref[0, k, :] = iks[k][0].astype(jnp.int32) + b * M
        wts_ref[0, k, :] = (recips[k] / norm)[0]

    @pl.when(j == 0)
    def _():
        w1a = w1_ref[:kf_ref.shape[2], :]
        g_ref[0] = jax.lax.dot_general(
            kf_ref[0].astype(jnp.bfloat16), w1a.astype(jnp.bfloat16),
            (((1,), (0,)), ((), ())),
            preferred_element_type=jnp.float32)


def _mlp_body(x1a_ref, uf_ref, w1_ref, b1_ref, g1_ref, be1_ref, w2_ref,
              b2_ref, g2_ref, be2_ref, out_ref):
    uf = uf_ref[0]
    C2 = x1a_ref.shape[2]
    w1b = w1_ref[C2:, :]
    x = (x1a_ref[0]
         + jax.lax.dot_general(uf.astype(jnp.bfloat16),
                               w1b.astype(jnp.bfloat16),
                               (((1,), (0,)), ((), ())),
                               preferred_element_type=jnp.float32)
         + b1_ref[0][None, :])
    x = x / jnp.sqrt(1.0 + _EPS_BN) * g1_ref[0][None, :] + be1_ref[0][None, :]
    x = jnp.maximum(x, 0.0)
    x = (jax.lax.dot_general(x.astype(jnp.bfloat16),
                             w2_ref[...].astype(jnp.bfloat16),
                             (((1,), (0,)), ((), ())),
                             preferred_element_type=jnp.float32)
         + b2_ref[0][None, :])
    x = x / jnp.sqrt(1.0 + _EPS_BN) * g2_ref[0][None, :] + be2_ref[0][None, :]
    out_ref[0] = jnp.maximum(x, 0.0)


def _make_sc_interp(B, n, H1):
    qw = B * n // _NW
    wpb = _NW // B
    nch = qw // _CH
    mesh = plsc.VectorSubcoreMesh(core_axis_name="c", subcore_axis_name="s")

    dnums = lax.GatherDimensionNumbers(
        offset_dims=(), collapsed_slice_dims=(0,), start_index_map=(0,))

    def _splat(v, c):
        idx = jnp.full((16, 1), c, jnp.int32)
        return lax.gather(v, idx, dnums, (1,),
                          mode=lax.GatherScatterMode.PROMISE_IN_BOUNDS)

    npair = nch // 2

    @functools.partial(
        pl.kernel, mesh=mesh,
        out_type=jax.ShapeDtypeStruct((B, n, H1), jnp.float32),
        scratch_types=[
            pltpu.VMEM((qw,), jnp.int32),
            pltpu.VMEM((qw,), jnp.int32),
            pltpu.VMEM((qw,), jnp.int32),
            pltpu.VMEM((qw,), jnp.float32),
            pltpu.VMEM((qw,), jnp.float32),
            pltpu.VMEM((qw,), jnp.float32),
            pltpu.VMEM((_CH, H1), jnp.float32),
            pltpu.VMEM((_CH, H1), jnp.float32),
            pltpu.VMEM((_CH, H1), jnp.float32),
            pltpu.VMEM((_CH, H1), jnp.float32),
            pltpu.VMEM((_CH, H1), jnp.float32),
            pltpu.VMEM((_CH, H1), jnp.float32),
            pltpu.VMEM((_CH, H1), jnp.float32),
            pltpu.SemaphoreType.DMA,
            pltpu.SemaphoreType.DMA,
        ],
    )
    def sc_interp(fidx_hbm, wts_hbm, g_hbm, out_hbm, i0_v, i1_v, i2_v,
                  w0_v, w1_v, w2_v, ra0, ra1, ra2, rb0, rb1, rb2, out_v,
                  sem0, sem1):
        wid = lax.axis_index("s") * 2 + lax.axis_index("c")
        b = wid // wpb
        q0 = (wid % wpb) * qw
        idx_vs = (i0_v, i1_v, i2_v)
        w_vs = (w0_v, w1_v, w2_v)
        slot_a = (ra0, ra1, ra2)
        slot_b = (rb0, rb1, rb2)

        for k in range(3):
            base = (b * 3 + k) * n + q0
            pltpu.sync_copy(fidx_hbm.at[pl.ds(base, qw)], idx_vs[k])
            pltpu.sync_copy(wts_hbm.at[pl.ds(base, qw)], w_vs[k])

        def fire(t, slot, sem):
            for k in range(3):
                pltpu.async_copy(
                    g_hbm.at[idx_vs[k].at[pl.ds(t * _CH, _CH)]], slot[k],
                    sem)

        def drain(slot, sem):
            for k in range(3):
                pltpu.make_async_copy(g_hbm.at[pl.ds(0, _CH)], slot[k],
                                      sem).wait()

        def compute(t, slot):
            for g in range(_CH // 16):
                wg = [w_vs[k][pl.ds(t * _CH + 16 * g, 16)] for k in range(3)]

                def one_query(c2, wgs):
                    c = 16 * g + c2
                    ws = [_splat(wgs[k], c2) for k in range(3)]
                    for f in range(H1 // 16):
                        sl = pl.ds(16 * f, 16)
                        acc = ws[0] * slot[0][c, sl]
                        acc = acc + ws[1] * slot[1][c, sl]
                        acc = acc + ws[2] * slot[2][c, sl]
                        out_v[c, sl] = acc
                    return wgs

                lax.fori_loop(0, 16, one_query, tuple(wg), unroll=4)
            pltpu.sync_copy(out_v, out_hbm.at[b].at[pl.ds(q0 + t * _CH,
                                                          _CH)])

        fire(0, slot_a, sem0)

        def pair(p, _):
            t0 = 2 * p
            fire(t0 + 1, slot_b, sem1)
            drain(slot_a, sem0)
            compute(t0, slot_a)

            @pl.when(p + 1 < npair)
            def _():
                fire(t0 + 2, slot_a, sem0)

            drain(slot_b, sem1)
            compute(t0 + 1, slot_b)
            return _

        lax.fori_loop(0, npair, pair, None)

    return sc_interp


@jax.jit
def kernel(unknown, known, unknow_feats, known_feats, grouped_xyz, inds,
           W1, b1, gamma1, beta1, W2, b2, gamma2, beta2):
    del grouped_xyz, inds
    B = unknown.shape[0]
    halves = []
    for hb in range(4):
        s = slice(hb * (B // 4), (hb + 1) * (B // 4))
        halves.append(_pipeline(
            unknown[s], known[s], unknow_feats[s], known_feats[s],
            W1, b1, gamma1, beta1, W2, b2, gamma2, beta2))
    return jnp.concatenate(halves, axis=0)


def _pipeline(unknown, known, unknow_feats, known_feats,
              W1, b1, gamma1, beta1, W2, b2, gamma2, beta2):
    B, n, _ = unknown.shape
    m = known.shape[1]
    C1 = unknow_feats.shape[2]
    C2 = known_feats.shape[2]
    H1 = W1.shape[1]
    H2 = W2.shape[1]
    unknownT = jnp.swapaxes(unknown, 1, 2)

    nblk = _NBLK
    grid = (B, n // nblk)

    def row(p):
        return p.reshape(1, -1)

    fidx, wts, G = pl.pallas_call(
        _nn_body,
        grid=grid,
        in_specs=[
            pl.BlockSpec((1, 3, nblk), lambda b, j: (b, 0, j)),
            pl.BlockSpec((1, m, 3), lambda b, j: (b, 0, 0)),
            pl.BlockSpec((1, m, C2), lambda b, j: (b, 0, 0)),
            pl.BlockSpec((C1 + C2, H1), lambda b, j: (0, 0)),
        ],
        out_specs=[
            pl.BlockSpec((1, 3, nblk), lambda b, j: (b, 0, j)),
            pl.BlockSpec((1, 3, nblk), lambda b, j: (b, 0, j)),
            pl.BlockSpec((1, m, H1), lambda b, j: (b, 0, 0)),
        ],
        out_shape=[
            jax.ShapeDtypeStruct((B, 3, n), jnp.int32),
            jax.ShapeDtypeStruct((B, 3, n), jnp.float32),
            jax.ShapeDtypeStruct((B, m, H1), jnp.float32),
        ],
    )(unknownT, known, known_feats, W1)

    x1a = _make_sc_interp(B, n, H1)(fidx.reshape(-1), wts.reshape(-1),
                                    G.reshape(B * m, H1))

    out = pl.pallas_call(
        _mlp_body,
        grid=grid,
        in_specs=[
            pl.BlockSpec((1, nblk, H1), lambda b, j: (b, j, 0)),
            pl.BlockSpec((1, nblk, C1), lambda b, j: (b, j, 0)),
            pl.BlockSpec((C1 + C2, H1), lambda b, j: (0, 0)),
            pl.BlockSpec((1, H1), lambda b, j: (0, 0)),
            pl.BlockSpec((1, H1), lambda b, j: (0, 0)),
            pl.BlockSpec((1, H1), lambda b, j: (0, 0)),
            pl.BlockSpec((H1, H2), lambda b, j: (0, 0)),
            pl.BlockSpec((1, H2), lambda b, j: (0, 0)),
            pl.BlockSpec((1, H2), lambda b, j: (0, 0)),
            pl.BlockSpec((1, H2), lambda b, j: (0, 0)),
        ],
        out_specs=pl.BlockSpec((1, nblk, H2), lambda b, j: (b, j, 0)),
        out_shape=jax.ShapeDtypeStruct((B, n, H2), jnp.float32),
    )(x1a, unknow_feats, W1, row(b1), row(gamma1), row(beta1), W2, row(b2),
      row(gamma2), row(beta2))
    return out


# speedup vs baseline: 1.0512x; 1.0512x over previous
import functools
import jax
import jax.numpy as jnp
from jax import lax
from jax.experimental import pallas as pl
from jax.experimental.pallas import tpu as pltpu
from jax.experimental.pallas import tpu_sc as plsc

_NBLK = 512
_EPS_BN = 1e-3
_BIG = 3.0e38
_NW = 32
_CH = 64


def _nn_body(ut_ref, kn_ref, kf_ref, w1_ref, idx_ref, wts_ref, g_ref):
    b = pl.program_id(0)
    j = pl.program_id(1)
    ut = ut_ref[0]
    kn = kn_ref[0]
    M = kn.shape[0]
    N = ut.shape[1]

    un2 = jnp.sum(ut * ut, axis=0, keepdims=True)
    kn2 = jnp.sum(kn * kn, axis=1, keepdims=True)
    cross = jax.lax.dot_general(
        kn.astype(jnp.bfloat16), ut.astype(jnp.bfloat16),
        (((1,), (0,)), ((), ())),
        preferred_element_type=jnp.float32)
    d2 = jnp.maximum(kn2 + un2 - 2.0 * cross, 0.0)

    iota_f = lax.broadcasted_iota(jnp.int32, (M, N), 0).astype(jnp.float32)
    d2 = jnp.where(d2 == 0.0, iota_f * 1e-30, d2)

    iota_m = lax.broadcasted_iota(jnp.int32, (2, M), 1)
    hilo = jnp.where(lax.broadcasted_iota(jnp.int32, (2, M), 0) == 0,
                     iota_m // 64, iota_m % 64).astype(jnp.bfloat16)

    recips = []
    iks = []
    d2w = d2
    for _ in range(3):
        mk = jnp.min(d2w, axis=0, keepdims=True)
        recips.append(1.0 / (mk + 1e-8))
        sel = d2w == mk
        hl = jax.lax.dot_general(
            hilo, sel.astype(jnp.bfloat16), (((1,), (0,)), ((), ())),
            preferred_element_type=jnp.float32)
        iks.append(hl[0:1, :] * 64.0 + hl[1:2, :])
        d2w = jnp.where(sel, _BIG, d2w)

    norm = recips[0] + recips[1] + recips[2]
    for k in range(3):
        idx_ref[0, k, :] = iks[k][0].astype(jnp.int32) + b * M
        wts_ref[0, k, :] = (recips[k] / norm)[0]

    @pl.when(j == 0)
    def _():
        w1a = w1_ref[:kf_ref.shape[2], :]
        g_ref[0] = jax.lax.dot_general(
            kf_ref[0].astype(jnp.bfloat16), w1a.astype(jnp.bfloat16),
            (((1,), (0,)), ((), ())),
            preferred_element_type=jnp.float32)


def _mlp_body(x1a_ref, uf_ref, w1_ref, b1_ref, g1_ref, be1_ref, w2_ref,
              b2_ref, g2_ref, be2_ref, out_ref):
    uf = uf_ref[0]
    C2 = x1a_ref.shape[2]
    w1b = w1_ref[C2:, :]
    x = (x1a_ref[0]
         + jax.lax.dot_general(uf.astype(jnp.bfloat16),
                               w1b.astype(jnp.bfloat16),
                               (((1,), (0,)), ((), ())),
                               preferred_element_type=jnp.float32)
         + b1_ref[0][None, :])
    x = x / jnp.sqrt(1.0 + _EPS_BN) * g1_ref[0][None, :] + be1_ref[0][None, :]
    x = jnp.maximum(x, 0.0)
    x = (jax.lax.dot_general(x.astype(jnp.bfloat16),
                             w2_ref[...].astype(jnp.bfloat16),
                             (((1,), (0,)), ((), ())),
                             preferred_element_type=jnp.float32)
         + b2_ref[0][None, :])
    x = x / jnp.sqrt(1.0 + _EPS_BN) * g2_ref[0][None, :] + be2_ref[0][None, :]
    out_ref[0] = jnp.maximum(x, 0.0)


def _make_sc_interp(B, n, H1):
    qw = B * n // _NW
    wpb = _NW // B
    nch = qw // _CH
    mesh = plsc.VectorSubcoreMesh(core_axis_name="c", subcore_axis_name="s")

    dnums = lax.GatherDimensionNumbers(
        offset_dims=(), collapsed_slice_dims=(0,), start_index_map=(0,))

    def _splat(v, c):
        idx = jnp.full((16, 1), c, jnp.int32)
        return lax.gather(v, idx, dnums, (1,),
                          mode=lax.GatherScatterMode.PROMISE_IN_BOUNDS)

    npair = nch // 2

    @functools.partial(
        pl.kernel, mesh=mesh,
        out_type=jax.ShapeDtypeStruct((B, n, H1), jnp.float32),
        scratch_types=[
            pltpu.VMEM((qw,), jnp.int32),
            pltpu.VMEM((qw,), jnp.int32),
            pltpu.VMEM((qw,), jnp.int32),
            pltpu.VMEM((qw,), jnp.float32),
            pltpu.VMEM((qw,), jnp.float32),
            pltpu.VMEM((qw,), jnp.float32),
            pltpu.VMEM((_CH, H1), jnp.float32),
            pltpu.VMEM((_CH, H1), jnp.float32),
            pltpu.VMEM((_CH, H1), jnp.float32),
            pltpu.VMEM((_CH, H1), jnp.float32),
            pltpu.VMEM((_CH, H1), jnp.float32),
            pltpu.VMEM((_CH, H1), jnp.float32),
            pltpu.VMEM((_CH, H1), jnp.float32),
            pltpu.SemaphoreType.DMA,
            pltpu.SemaphoreType.DMA,
        ],
    )
    def sc_interp(fidx_hbm, wts_hbm, g_hbm, out_hbm, i0_v, i1_v, i2_v,
                  w0_v, w1_v, w2_v, ra0, ra1, ra2, rb0, rb1, rb2, out_v,
                  sem0, sem1):
        wid = lax.axis_index("s") * 2 + lax.axis_index("c")
        b = wid // wpb
        q0 = (wid % wpb) * qw
        idx_vs = (i0_v, i1_v, i2_v)
        w_vs = (w0_v, w1_v, w2_v)
        slot_a = (ra0, ra1, ra2)
        slot_b = (rb0, rb1, rb2)

        wcps = [pltpu.async_copy(
            wts_hbm.at[pl.ds((b * 3 + k) * n + q0, qw)], w_vs[k], sem1)
            for k in range(3)]
        for k in range(3):
            pltpu.sync_copy(fidx_hbm.at[pl.ds((b * 3 + k) * n + q0, qw)],
                            idx_vs[k])

        def fire(t, slot, sem):
            for k in range(3):
                pltpu.async_copy(
                    g_hbm.at[idx_vs[k].at[pl.ds(t * _CH, _CH)]], slot[k],
                    sem)

        def drain(slot, sem):
            for k in range(3):
                pltpu.make_async_copy(g_hbm.at[pl.ds(0, _CH)], slot[k],
                                      sem).wait()

        def compute(t, slot):
            def group(g, _):
                wg = [w_vs[k][pl.ds(t * _CH + 16 * g, 16)]
                      for k in range(3)]

                def one_query(c2, wgs):
                    c = 16 * g + c2
                    ws = [_splat(wgs[k], c2) for k in range(3)]
                    for f in range(H1 // 16):
                        sl = pl.ds(16 * f, 16)
                        acc = ws[0] * slot[0][c, sl]
                        acc = acc + ws[1] * slot[1][c, sl]
                        acc = acc + ws[2] * slot[2][c, sl]
                        out_v[c, sl] = acc
                    return wgs

                lax.fori_loop(0, 16, one_query, tuple(wg), unroll=8)
                return _

            lax.fori_loop(0, _CH // 16, group, None)
            pltpu.sync_copy(out_v, out_hbm.at[b].at[pl.ds(q0 + t * _CH,
                                                          _CH)])

        fire(0, slot_a, sem0)
        for cp in wcps:
            cp.wait()

        def pair(p, _):
            t0 = 2 * p
            fire(t0 + 1, slot_b, sem1)
            drain(slot_a, sem0)
            compute(t0, slot_a)

            @pl.when(p + 1 < npair)
            def _():
                fire(t0 + 2, slot_a, sem0)

            drain(slot_b, sem1)
            compute(t0 + 1, slot_b)
            return _

        lax.fori_loop(0, npair, pair, None)

    return sc_interp


@jax.jit
def kernel(unknown, known, unknow_feats, known_feats, grouped_xyz, inds,
           W1, b1, gamma1, beta1, W2, b2, gamma2, beta2):
    del grouped_xyz, inds
    B = unknown.shape[0]
    halves = []
    for hb in range(2):
        s = slice(hb * (B // 2), (hb + 1) * (B // 2))
        halves.append(_pipeline(
            unknown[s], known[s], unknow_feats[s], known_feats[s],
            W1, b1, gamma1, beta1, W2, b2, gamma2, beta2))
    return jnp.concatenate(halves, axis=0)


def _pipeline(unknown, known, unknow_feats, known_feats,
              W1, b1, gamma1, beta1, W2, b2, gamma2, beta2):
    B, n, _ = unknown.shape
    m = known.shape[1]
    C1 = unknow_feats.shape[2]
    C2 = known_feats.shape[2]
    H1 = W1.shape[1]
    H2 = W2.shape[1]
    unknownT = jnp.swapaxes(unknown, 1, 2)

    nblk = _NBLK
    grid = (B, n // nblk)

    def row(p):
        return p.reshape(1, -1)

    fidx, wts, G = pl.pallas_call(
        _nn_body,
        grid=grid,
        in_specs=[
            pl.BlockSpec((1, 3, nblk), lambda b, j: (b, 0, j)),
            pl.BlockSpec((1, m, 3), lambda b, j: (b, 0, 0)),
            pl.BlockSpec((1, m, C2), lambda b, j: (b, 0, 0)),
            pl.BlockSpec((C1 + C2, H1), lambda b, j: (0, 0)),
        ],
        out_specs=[
            pl.BlockSpec((1, 3, nblk), lambda b, j: (b, 0, j)),
            pl.BlockSpec((1, 3, nblk), lambda b, j: (b, 0, j)),
            pl.BlockSpec((1, m, H1), lambda b, j: (b, 0, 0)),
        ],
        out_shape=[
            jax.ShapeDtypeStruct((B, 3, n), jnp.int32),
            jax.ShapeDtypeStruct((B, 3, n), jnp.float32),
            jax.ShapeDtypeStruct((B, m, H1), jnp.float32),
        ],
    )(unknownT, known, known_feats, W1)

    x1a = _make_sc_interp(B, n, H1)(fidx.reshape(-1), wts.reshape(-1),
                                    G.reshape(B * m, H1))

    out = pl.pallas_call(
        _mlp_body,
        grid=grid,
        in_specs=[
            pl.BlockSpec((1, nblk, H1), lambda b, j: (b, j, 0)),
            pl.BlockSpec((1, nblk, C1), lambda b, j: (b, j, 0)),
            pl.BlockSpec((C1 + C2, H1), lambda b, j: (0, 0)),
            pl.BlockSpec((1, H1), lambda b, j: (0, 0)),
            pl.BlockSpec((1, H1), lambda b, j: (0, 0)),
            pl.BlockSpec((1, H1), lambda b, j: (0, 0)),
            pl.BlockSpec((H1, H2), lambda b, j: (0, 0)),
            pl.BlockSpec((1, H2), lambda b, j: (0, 0)),
            pl.BlockSpec((1, H2), lambda b, j: (0, 0)),
            pl.BlockSpec((1, H2), lambda b, j: (0, 0)),
        ],
        out_specs=pl.BlockSpec((1, nblk, H2), lambda b, j: (b, j, 0)),
        out_shape=jax.ShapeDtypeStruct((B, n, H2), jnp.float32),
    )(x1a, unknow_feats, W1, row(b1), row(gamma1), row(beta1), W2, row(b2),
      row(gamma2), row(beta2))
    return out


# speedup vs baseline: 1.0596x; 1.0080x over previous
import functools
import jax
import jax.numpy as jnp
from jax import lax
from jax.experimental import pallas as pl
from jax.experimental.pallas import tpu as pltpu
from jax.experimental.pallas import tpu_sc as plsc

_NBLK = 512
_EPS_BN = 1e-3
_BIG = 3.0e38
_NW = 32
_CH = 64


def _nn_body(ut_ref, kn_ref, kf_ref, w1_ref, idx_ref, wts_ref, g_ref):
    b = pl.program_id(0)
    j = pl.program_id(1)
    ut = ut_ref[0]
    kn = kn_ref[0]
    M = kn.shape[0]
    N = ut.shape[1]

    un2 = jnp.sum(ut * ut, axis=0, keepdims=True)
    kn2 = jnp.sum(kn * kn, axis=1, keepdims=True)
    cross = jax.lax.dot_general(
        kn.astype(jnp.bfloat16), ut.astype(jnp.bfloat16),
        (((1,), (0,)), ((), ())),
        preferred_element_type=jnp.float32)
    d2 = jnp.maximum(kn2 + un2 - 2.0 * cross, 0.0)

    iota_f = lax.broadcasted_iota(jnp.int32, (M, N), 0).astype(jnp.float32)
    d2 = jnp.where(d2 == 0.0, iota_f * 1e-30, d2)

    iota_m = lax.broadcasted_iota(jnp.int32, (2, M), 1)
    hilo = jnp.where(lax.broadcasted_iota(jnp.int32, (2, M), 0) == 0,
                     iota_m // 64, iota_m % 64).astype(jnp.bfloat16)

    recips = []
    iks = []
    d2w = d2
    for _ in range(3):
        mk = jnp.min(d2w, axis=0, keepdims=True)
        recips.append(1.0 / (mk + 1e-8))
        sel = d2w == mk
        hl = jax.lax.dot_general(
            hilo, sel.astype(jnp.bfloat16), (((1,), (0,)), ((), ())),
            preferred_element_type=jnp.float32)
        iks.append(hl[0:1, :] * 64.0 + hl[1:2, :])
        d2w = jnp.where(sel, _BIG, d2w)

    norm = recips[0] + recips[1] + recips[2]
    for k in range(3):
        idx_ref[0, k, :] = iks[k][0].astype(jnp.int32) + b * M
        wts_ref[0, k, :] = (recips[k] / norm)[0]

    @pl.when(j == 0)
    def _():
        w1a = w1_ref[:kf_ref.shape[2], :]
        g_ref[0] = jax.lax.dot_general(
            kf_ref[0].astype(jnp.bfloat16), w1a.astype(jnp.bfloat16),
            (((1,), (0,)), ((), ())),
            preferred_element_type=jnp.float32)


def _mlp_body(x1a_ref, uf_ref, w1_ref, b1_ref, g1_ref, be1_ref, w2_ref,
              b2_ref, g2_ref, be2_ref, out_ref):
    uf = uf_ref[0]
    C2 = x1a_ref.shape[2]
    w1b = w1_ref[C2:, :]
    x = (x1a_ref[0]
         + jax.lax.dot_general(uf.astype(jnp.bfloat16),
                               w1b.astype(jnp.bfloat16),
                               (((1,), (0,)), ((), ())),
                               preferred_element_type=jnp.float32)
         + b1_ref[0][None, :])
    x = x / jnp.sqrt(1.0 + _EPS_BN) * g1_ref[0][None, :] + be1_ref[0][None, :]
    x = jnp.maximum(x, 0.0)
    x = (jax.lax.dot_general(x.astype(jnp.bfloat16),
                             w2_ref[...].astype(jnp.bfloat16),
                             (((1,), (0,)), ((), ())),
                             preferred_element_type=jnp.float32)
         + b2_ref[0][None, :])
    x = x / jnp.sqrt(1.0 + _EPS_BN) * g2_ref[0][None, :] + be2_ref[0][None, :]
    out_ref[0] = jnp.maximum(x, 0.0)


def _make_sc_interp(B, n, H1):
    qw = B * n // _NW
    wpb = _NW // B
    nch = qw // _CH
    mesh = plsc.VectorSubcoreMesh(core_axis_name="c", subcore_axis_name="s")

    dnums = lax.GatherDimensionNumbers(
        offset_dims=(), collapsed_slice_dims=(0,), start_index_map=(0,))

    def _splat(v, c):
        idx = jnp.full((16, 1), c, jnp.int32)
        return lax.gather(v, idx, dnums, (1,),
                          mode=lax.GatherScatterMode.PROMISE_IN_BOUNDS)

    npair = nch // 2

    @functools.partial(
        pl.kernel, mesh=mesh,
        out_type=jax.ShapeDtypeStruct((B, n, H1), jnp.float32),
        scratch_types=[
            pltpu.VMEM((qw,), jnp.int32),
            pltpu.VMEM((qw,), jnp.int32),
            pltpu.VMEM((qw,), jnp.int32),
            pltpu.VMEM((qw,), jnp.float32),
            pltpu.VMEM((qw,), jnp.float32),
            pltpu.VMEM((qw,), jnp.float32),
            pltpu.VMEM((_CH, H1), jnp.float32),
            pltpu.VMEM((_CH, H1), jnp.float32),
            pltpu.VMEM((_CH, H1), jnp.float32),
            pltpu.VMEM((_CH, H1), jnp.float32),
            pltpu.VMEM((_CH, H1), jnp.float32),
            pltpu.VMEM((_CH, H1), jnp.float32),
            pltpu.VMEM((_CH, H1), jnp.float32),
            pltpu.SemaphoreType.DMA,
            pltpu.SemaphoreType.DMA,
        ],
    )
    def sc_interp(fidx_hbm, wts_hbm, g_hbm, out_hbm, i0_v, i1_v, i2_v,
                  w0_v, w1_v, w2_v, ra0, ra1, ra2, rb0, rb1, rb2, out_v,
                  sem0, sem1):
        wid = lax.axis_index("s") * 2 + lax.axis_index("c")
        b = wid // wpb
        q0 = (wid % wpb) * qw
        idx_vs = (i0_v, i1_v, i2_v)
        w_vs = (w0_v, w1_v, w2_v)
        slot_a = (ra0, ra1, ra2)
        slot_b = (rb0, rb1, rb2)

        wcps = [pltpu.async_copy(
            wts_hbm.at[pl.ds((b * 3 + k) * n + q0, qw)], w_vs[k], sem1)
            for k in range(3)]
        for k in range(3):
            pltpu.sync_copy(fidx_hbm.at[pl.ds((b * 3 + k) * n + q0, qw)],
                            idx_vs[k])

        def fire(t, slot, sem):
            for k in range(3):
                pltpu.async_copy(
                    g_hbm.at[idx_vs[k].at[pl.ds(t * _CH, _CH)]], slot[k],
                    sem)

        def drain(slot, sem):
            for k in range(3):
                pltpu.make_async_copy(g_hbm.at[pl.ds(0, _CH)], slot[k],
                                      sem).wait()

        def compute(t, slot):
            for g in range(_CH // 16):
                wg = [w_vs[k][pl.ds(t * _CH + 16 * g, 16)]
                      for k in range(3)]

                def one_query(c2, wgs):
                    c = 16 * g + c2
                    ws = [_splat(wgs[k], c2) for k in range(3)]
                    for f in range(H1 // 16):
                        sl = pl.ds(16 * f, 16)
                        acc = ws[0] * slot[0][c, sl]
                        acc = acc + ws[1] * slot[1][c, sl]
                        acc = acc + ws[2] * slot[2][c, sl]
                        out_v[c, sl] = acc
                    return wgs

                lax.fori_loop(0, 16, one_query, tuple(wg), unroll=4)
            pltpu.sync_copy(out_v, out_hbm.at[b].at[pl.ds(q0 + t * _CH,
                                                          _CH)])

        fire(0, slot_a, sem0)
        for cp in wcps:
            cp.wait()

        def pair(p, _):
            t0 = 2 * p
            fire(t0 + 1, slot_b, sem1)
            drain(slot_a, sem0)
            compute(t0, slot_a)

            @pl.when(p + 1 < npair)
            def _():
                fire(t0 + 2, slot_a, sem0)

            drain(slot_b, sem1)
            compute(t0 + 1, slot_b)
            return _

        lax.fori_loop(0, npair, pair, None)

    return sc_interp


@jax.jit
def kernel(unknown, known, unknow_feats, known_feats, grouped_xyz, inds,
           W1, b1, gamma1, beta1, W2, b2, gamma2, beta2):
    del grouped_xyz, inds
    B = unknown.shape[0]
    halves = []
    for hb in range(2):
        s = slice(hb * (B // 2), (hb + 1) * (B // 2))
        halves.append(_pipeline(
            unknown[s], known[s], unknow_feats[s], known_feats[s],
            W1, b1, gamma1, beta1, W2, b2, gamma2, beta2))
    return jnp.concatenate(halves, axis=0)


def _pipeline(unknown, known, unknow_feats, known_feats,
              W1, b1, gamma1, beta1, W2, b2, gamma2, beta2):
    B, n, _ = unknown.shape
    m = known.shape[1]
    C1 = unknow_feats.shape[2]
    C2 = known_feats.shape[2]
    H1 = W1.shape[1]
    H2 = W2.shape[1]
    unknownT = jnp.swapaxes(unknown, 1, 2)

    nblk = _NBLK
    grid = (B, n // nblk)

    def row(p):
        return p.reshape(1, -1)

    fidx, wts, G = pl.pallas_call(
        _nn_body,
        grid=grid,
        in_specs=[
            pl.BlockSpec((1, 3, nblk), lambda b, j: (b, 0, j)),
            pl.BlockSpec((1, m, 3), lambda b, j: (b, 0, 0)),
            pl.BlockSpec((1, m, C2), lambda b, j: (b, 0, 0)),
            pl.BlockSpec((C1 + C2, H1), lambda b, j: (0, 0)),
        ],
        out_specs=[
            pl.BlockSpec((1, 3, nblk), lambda b, j: (b, 0, j)),
            pl.BlockSpec((1, 3, nblk), lambda b, j: (b, 0, j)),
            pl.BlockSpec((1, m, H1), lambda b, j: (b, 0, 0)),
        ],
        out_shape=[
            jax.ShapeDtypeStruct((B, 3, n), jnp.int32),
            jax.ShapeDtypeStruct((B, 3, n), jnp.float32),
            jax.ShapeDtypeStruct((B, m, H1), jnp.float32),
        ],
    )(unknownT, known, known_feats, W1)

    x1a = _make_sc_interp(B, n, H1)(fidx.reshape(-1), wts.reshape(-1),
                                    G.reshape(B * m, H1))

    out = pl.pallas_call(
        _mlp_body,
        grid=grid,
        in_specs=[
            pl.BlockSpec((1, nblk, H1), lambda b, j: (b, j, 0)),
            pl.BlockSpec((1, nblk, C1), lambda b, j: (b, j, 0)),
            pl.BlockSpec((C1 + C2, H1), lambda b, j: (0, 0)),
            pl.BlockSpec((1, H1), lambda b, j: (0, 0)),
            pl.BlockSpec((1, H1), lambda b, j: (0, 0)),
            pl.BlockSpec((1, H1), lambda b, j: (0, 0)),
            pl.BlockSpec((H1, H2), lambda b, j: (0, 0)),
            pl.BlockSpec((1, H2), lambda b, j: (0, 0)),
            pl.BlockSpec((1, H2), lambda b, j: (0, 0)),
            pl.BlockSpec((1, H2), lambda b, j: (0, 0)),
        ],
        out_specs=pl.BlockSpec((1, nblk, H2), lambda b, j: (b, j, 0)),
        out_shape=jax.ShapeDtypeStruct((B, n, H2), jnp.float32),
    )(x1a, unknow_feats, W1, row(b1), row(gamma1), row(beta1), W2, row(b2),
      row(gamma2), row(beta2))
    return out


# speedup vs baseline: 1.0950x; 1.0334x over previous
import functools
import jax
import jax.numpy as jnp
from jax import lax
from jax.experimental import pallas as pl
from jax.experimental.pallas import tpu as pltpu
from jax.experimental.pallas import tpu_sc as plsc

_NBLK = 1024
_EPS_BN = 1e-3
_BIG = 3.0e38
_NW = 32
_CH = 64


def _nn_body(ut_ref, kn_ref, kf_ref, w1_ref, idx_ref, wts_ref, g_ref):
    b = pl.program_id(0)
    j = pl.program_id(1)
    ut = ut_ref[0]
    kn = kn_ref[0]
    M = kn.shape[0]
    N = ut.shape[1]

    un2 = jnp.sum(ut * ut, axis=0, keepdims=True)
    kn2 = jnp.sum(kn * kn, axis=1, keepdims=True)
    cross = jax.lax.dot_general(
        kn.astype(jnp.bfloat16), ut.astype(jnp.bfloat16),
        (((1,), (0,)), ((), ())),
        preferred_element_type=jnp.float32)
    d2 = jnp.maximum(kn2 + un2 - 2.0 * cross, 0.0)

    iota_f = lax.broadcasted_iota(jnp.int32, (M, N), 0).astype(jnp.float32)
    d2 = jnp.where(d2 == 0.0, iota_f * 1e-30, d2)

    iota_m = lax.broadcasted_iota(jnp.int32, (2, M), 1)
    hilo = jnp.where(lax.broadcasted_iota(jnp.int32, (2, M), 0) == 0,
                     iota_m // 64, iota_m % 64).astype(jnp.bfloat16)

    recips = []
    iks = []
    d2w = d2
    for _ in range(3):
        mk = jnp.min(d2w, axis=0, keepdims=True)
        recips.append(1.0 / (mk + 1e-8))
        sel = d2w == mk
        hl = jax.lax.dot_general(
            hilo, sel.astype(jnp.bfloat16), (((1,), (0,)), ((), ())),
            preferred_element_type=jnp.float32)
        iks.append(hl[0:1, :] * 64.0 + hl[1:2, :])
        d2w = jnp.where(sel, _BIG, d2w)

    norm = recips[0] + recips[1] + recips[2]
    for k in range(3):
        idx_ref[0, k, :] = iks[k][0].astype(jnp.int32) + b * M
        wts_ref[0, k, :] = (recips[k] / norm)[0]

    @pl.when(j == 0)
    def _():
        w1a = w1_ref[:kf_ref.shape[2], :]
        g_ref[0] = jax.lax.dot_general(
            kf_ref[0].astype(jnp.bfloat16), w1a.astype(jnp.bfloat16),
            (((1,), (0,)), ((), ())),
            preferred_element_type=jnp.float32)


def _mlp_body(x1a_ref, uf_ref, w1_ref, b1_ref, g1_ref, be1_ref, w2_ref,
              b2_ref, g2_ref, be2_ref, out_ref):
    uf = uf_ref[0]
    C2 = x1a_ref.shape[2]
    w1b = w1_ref[C2:, :]
    x = (x1a_ref[0]
         + jax.lax.dot_general(uf.astype(jnp.bfloat16),
                               w1b.astype(jnp.bfloat16),
                               (((1,), (0,)), ((), ())),
                               preferred_element_type=jnp.float32)
         + b1_ref[0][None, :])
    x = x / jnp.sqrt(1.0 + _EPS_BN) * g1_ref[0][None, :] + be1_ref[0][None, :]
    x = jnp.maximum(x, 0.0)
    x = (jax.lax.dot_general(x.astype(jnp.bfloat16),
                             w2_ref[...].astype(jnp.bfloat16),
                             (((1,), (0,)), ((), ())),
                             preferred_element_type=jnp.float32)
         + b2_ref[0][None, :])
    x = x / jnp.sqrt(1.0 + _EPS_BN) * g2_ref[0][None, :] + be2_ref[0][None, :]
    out_ref[0] = jnp.maximum(x, 0.0)


def _make_sc_interp(B, n, H1):
    qw = B * n // _NW
    wpb = _NW // B
    nch = qw // _CH
    mesh = plsc.VectorSubcoreMesh(core_axis_name="c", subcore_axis_name="s")

    dnums = lax.GatherDimensionNumbers(
        offset_dims=(), collapsed_slice_dims=(0,), start_index_map=(0,))

    def _splat(v, c):
        idx = jnp.full((16, 1), c, jnp.int32)
        return lax.gather(v, idx, dnums, (1,),
                          mode=lax.GatherScatterMode.PROMISE_IN_BOUNDS)

    npair = nch // 2

    @functools.partial(
        pl.kernel, mesh=mesh,
        out_type=jax.ShapeDtypeStruct((B, n, H1), jnp.float32),
        scratch_types=[
            pltpu.VMEM((qw,), jnp.int32),
            pltpu.VMEM((qw,), jnp.int32),
            pltpu.VMEM((qw,), jnp.int32),
            pltpu.VMEM((qw,), jnp.float32),
            pltpu.VMEM((qw,), jnp.float32),
            pltpu.VMEM((qw,), jnp.float32),
            pltpu.VMEM((_CH, H1), jnp.float32),
            pltpu.VMEM((_CH, H1), jnp.float32),
            pltpu.VMEM((_CH, H1), jnp.float32),
            pltpu.VMEM((_CH, H1), jnp.float32),
            pltpu.VMEM((_CH, H1), jnp.float32),
            pltpu.VMEM((_CH, H1), jnp.float32),
            pltpu.VMEM((_CH, H1), jnp.float32),
            pltpu.SemaphoreType.DMA,
            pltpu.SemaphoreType.DMA,
        ],
    )
    def sc_interp(fidx_hbm, wts_hbm, g_hbm, out_hbm, i0_v, i1_v, i2_v,
                  w0_v, w1_v, w2_v, ra0, ra1, ra2, rb0, rb1, rb2, out_v,
                  sem0, sem1):
        wid = lax.axis_index("s") * 2 + lax.axis_index("c")
        b = wid // wpb
        q0 = (wid % wpb) * qw
        idx_vs = (i0_v, i1_v, i2_v)
        w_vs = (w0_v, w1_v, w2_v)
        slot_a = (ra0, ra1, ra2)
        slot_b = (rb0, rb1, rb2)

        wcps = [pltpu.async_copy(
            wts_hbm.at[pl.ds((b * 3 + k) * n + q0, qw)], w_vs[k], sem1)
            for k in range(3)]
        for k in range(3):
            pltpu.sync_copy(fidx_hbm.at[pl.ds((b * 3 + k) * n + q0, qw)],
                            idx_vs[k])

        def fire(t, slot, sem):
            for k in range(3):
                pltpu.async_copy(
                    g_hbm.at[idx_vs[k].at[pl.ds(t * _CH, _CH)]], slot[k],
                    sem)

        def drain(slot, sem):
            for k in range(3):
                pltpu.make_async_copy(g_hbm.at[pl.ds(0, _CH)], slot[k],
                                      sem).wait()

        def compute(t, slot):
            for g in range(_CH // 16):
                wg = [w_vs[k][pl.ds(t * _CH + 16 * g, 16)]
                      for k in range(3)]

                def one_query(c2, wgs):
                    c = 16 * g + c2
                    ws = [_splat(wgs[k], c2) for k in range(3)]
                    for f in range(H1 // 16):
                        sl = pl.ds(16 * f, 16)
                        acc = ws[0] * slot[0][c, sl]
                        acc = acc + ws[1] * slot[1][c, sl]
                        acc = acc + ws[2] * slot[2][c, sl]
                        out_v[c, sl] = acc
                    return wgs

                lax.fori_loop(0, 16, one_query, tuple(wg), unroll=4)
            pltpu.sync_copy(out_v, out_hbm.at[b].at[pl.ds(q0 + t * _CH,
                                                          _CH)])

        fire(0, slot_a, sem0)
        for cp in wcps:
            cp.wait()

        def pair(p, _):
            t0 = 2 * p
            fire(t0 + 1, slot_b, sem1)
            drain(slot_a, sem0)
            compute(t0, slot_a)

            @pl.when(p + 1 < npair)
            def _():
                fire(t0 + 2, slot_a, sem0)

            drain(slot_b, sem1)
            compute(t0 + 1, slot_b)
            return _

        lax.fori_loop(0, npair, pair, None)

    return sc_interp


@jax.jit
def kernel(unknown, known, unknow_feats, known_feats, grouped_xyz, inds,
           W1, b1, gamma1, beta1, W2, b2, gamma2, beta2):
    del grouped_xyz, inds
    B = unknown.shape[0]
    halves = []
    for hb in range(2):
        s = slice(hb * (B // 2), (hb + 1) * (B // 2))
        halves.append(_pipeline(
            unknown[s], known[s], unknow_feats[s], known_feats[s],
            W1, b1, gamma1, beta1, W2, b2, gamma2, beta2))
    return jnp.concatenate(halves, axis=0)


def _pipeline(unknown, known, unknow_feats, known_feats,
              W1, b1, gamma1, beta1, W2, b2, gamma2, beta2):
    B, n, _ = unknown.shape
    m = known.shape[1]
    C1 = unknow_feats.shape[2]
    C2 = known_feats.shape[2]
    H1 = W1.shape[1]
    H2 = W2.shape[1]
    unknownT = jnp.swapaxes(unknown, 1, 2)

    nblk = _NBLK
    grid = (B, n // nblk)

    def row(p):
        return p.reshape(1, -1)

    fidx, wts, G = pl.pallas_call(
        _nn_body,
        grid=grid,
        in_specs=[
            pl.BlockSpec((1, 3, nblk), lambda b, j: (b, 0, j)),
            pl.BlockSpec((1, m, 3), lambda b, j: (b, 0, 0)),
            pl.BlockSpec((1, m, C2), lambda b, j: (b, 0, 0)),
            pl.BlockSpec((C1 + C2, H1), lambda b, j: (0, 0)),
        ],
        out_specs=[
            pl.BlockSpec((1, 3, nblk), lambda b, j: (b, 0, j)),
            pl.BlockSpec((1, 3, nblk), lambda b, j: (b, 0, j)),
            pl.BlockSpec((1, m, H1), lambda b, j: (b, 0, 0)),
        ],
        out_shape=[
            jax.ShapeDtypeStruct((B, 3, n), jnp.int32),
            jax.ShapeDtypeStruct((B, 3, n), jnp.float32),
            jax.ShapeDtypeStruct((B, m, H1), jnp.float32),
        ],
    )(unknownT, known, known_feats, W1)

    x1a = _make_sc_interp(B, n, H1)(fidx.reshape(-1), wts.reshape(-1),
                                    G.reshape(B * m, H1))

    out = pl.pallas_call(
        _mlp_body,
        grid=grid,
        in_specs=[
            pl.BlockSpec((1, nblk, H1), lambda b, j: (b, j, 0)),
            pl.BlockSpec((1, nblk, C1), lambda b, j: (b, j, 0)),
            pl.BlockSpec((C1 + C2, H1), lambda b, j: (0, 0)),
            pl.BlockSpec((1, H1), lambda b, j: (0, 0)),
            pl.BlockSpec((1, H1), lambda b, j: (0, 0)),
            pl.BlockSpec((1, H1), lambda b, j: (0, 0)),
            pl.BlockSpec((H1, H2), lambda b, j: (0, 0)),
            pl.BlockSpec((1, H2), lambda b, j: (0, 0)),
            pl.BlockSpec((1, H2), lambda b, j: (0, 0)),
            pl.BlockSpec((1, H2), lambda b, j: (0, 0)),
        ],
        out_specs=pl.BlockSpec((1, nblk, H2), lambda b, j: (b, j, 0)),
        out_shape=jax.ShapeDtypeStruct((B, n, H2), jnp.float32),
    )(x1a, unknow_feats, W1, row(b1), row(gamma1), row(beta1), W2, row(b2),
      row(gamma2), row(beta2))
    return out
